# Initial kernel scaffold; baseline (speedup 1.0000x reference)
import functools

import jax
import jax.numpy as jnp
from jax import lax
from jax.experimental import pallas as pl
from jax.experimental.pallas import tpu as pltpu
from jax.experimental.pallas import tpu_sc as plsc

N = 10000
D = 128
E = 320000

NC = 2
NS = 16
NW = NC * NS
EPW = E // NW
CHUNK = 80
STEPS = EPW // CHUNK
RPT = N // NS
DEGW = 16

_BN_SCALE = 1.0 / (1.0 + 1e-5) ** 0.5

_sc_mesh = plsc.VectorSubcoreMesh(core_axis_name="c", subcore_axis_name="s")


@functools.partial(
    pl.kernel,
    out_type=jax.ShapeDtypeStruct((NC, N, DEGW), jnp.float32),
    mesh=_sc_mesh,
    scratch_types=[
        pltpu.VMEM((CHUNK,), jnp.int32),
        pltpu.VMEM((CHUNK, DEGW), jnp.float32),
        pltpu.VMEM((RPT, DEGW), jnp.float32),
        pltpu.VMEM_SHARED((N, DEGW), jnp.float32),
    ],
)
def _deg_sc(dst_hbm, ones_hbm, zeros_hbm, out_hbm, didx, ones_v, zb_v, acc):
    c = lax.axis_index("c")
    s = lax.axis_index("s")
    w = s * NC + c
    pltpu.sync_copy(ones_hbm, ones_v)
    pltpu.sync_copy(zeros_hbm, zb_v)
    pltpu.sync_copy(zb_v, acc.at[pl.ds(s * RPT, RPT)])
    plsc.subcore_barrier()

    def step(j, carry):
        base = w * EPW + j * CHUNK
        pltpu.sync_copy(dst_hbm.at[pl.ds(base, CHUNK)], didx)
        pltpu.sync_copy(ones_v, acc.at[didx], add=True)
        return carry

    lax.fori_loop(0, STEPS, step, 0)
    plsc.subcore_barrier()
    pltpu.sync_copy(acc.at[pl.ds(s * RPT, RPT)], zb_v)
    pltpu.sync_copy(zb_v, out_hbm.at[c, pl.ds(s * RPT, RPT)])


@functools.partial(
    pl.kernel,
    out_type=jax.ShapeDtypeStruct((NC, N, D), jnp.float32),
    mesh=_sc_mesh,
    scratch_types=[
        pltpu.VMEM((CHUNK,), jnp.int32),
        pltpu.VMEM((CHUNK,), jnp.int32),
        pltpu.VMEM((CHUNK, D), jnp.float32),
        pltpu.VMEM((RPT, D), jnp.float32),
        pltpu.SemaphoreType.DMA,
        pltpu.VMEM_SHARED((N, D), jnp.float32),
    ],
)
def _spmm_sc(y_hbm, src_hbm, dst_hbm, zeros_hbm, out_hbm,
             sidx, didx, rows, stage, sem, acc):
    c = lax.axis_index("c")
    s = lax.axis_index("s")
    w = s * NC + c
    pltpu.sync_copy(zeros_hbm, stage)
    pltpu.sync_copy(stage, acc.at[pl.ds(s * RPT, RPT)])
    plsc.subcore_barrier()

    def step(j, carry):
        base = w * EPW + j * CHUNK
        pltpu.sync_copy(src_hbm.at[pl.ds(base, CHUNK)], sidx)
        pltpu.sync_copy(dst_hbm.at[pl.ds(base, CHUNK)], didx)
        pltpu.async_copy(y_hbm.at[sidx], rows, sem).wait()
        pltpu.sync_copy(rows, acc.at[didx], add=True)
        return carry

    lax.fori_loop(0, STEPS, step, 0)
    plsc.subcore_barrier()
    pltpu.sync_copy(acc.at[pl.ds(s * RPT, RPT)], stage)
    pltpu.sync_copy(stage, out_hbm.at[c, pl.ds(s * RPT, RPT)])


_R = 1000


def _a_body(degp_ref, x_ref, w_ref, y_ref, dinv_ref):
    deg = 1.0 + degp_ref[0, :, 0:1] + degp_ref[1, :, 0:1]
    dinv = lax.rsqrt(deg)
    dinv_ref[...] = dinv
    y_ref[...] = jnp.dot(x_ref[...], w_ref[...],
                         preferred_element_type=jnp.float32) * dinv


def _b_body(p_ref, y_ref, dinv_ref, b_ref, g_ref, be_ref, w_ref, yn_ref):
    dinv = dinv_ref[...]
    z = (p_ref[0] + p_ref[1] + y_ref[...]) * dinv + b_ref[...]
    t = jnp.maximum(z * (g_ref[...] * _BN_SCALE) + be_ref[...], 0.0)
    yn_ref[...] = jnp.dot(t, w_ref[...],
                          preferred_element_type=jnp.float32) * dinv


def _c_body(p_ref, y_ref, dinv_ref, b_ref, o_ref):
    z = (p_ref[0] + p_ref[1] + y_ref[...]) * dinv_ref[...] + b_ref[...]
    m = jnp.max(z, axis=1, keepdims=True)
    lse = jnp.log(jnp.sum(jnp.exp(z - m), axis=1, keepdims=True)) + m
    o_ref[...] = z - lse


_a_call = pl.pallas_call(
    _a_body,
    grid=(N // _R,),
    in_specs=[
        pl.BlockSpec((NC, _R, DEGW), lambda i: (0, i, 0)),
        pl.BlockSpec((_R, D), lambda i: (i, 0)),
        pl.BlockSpec((D, D), lambda i: (0, 0)),
    ],
    out_specs=[
        pl.BlockSpec((_R, D), lambda i: (i, 0)),
        pl.BlockSpec((_R, 1), lambda i: (i, 0)),
    ],
    out_shape=[
        jax.ShapeDtypeStruct((N, D), jnp.float32),
        jax.ShapeDtypeStruct((N, 1), jnp.float32),
    ],
)

_b_call = pl.pallas_call(
    _b_body,
    grid=(N // _R,),
    in_specs=[
        pl.BlockSpec((NC, _R, D), lambda i: (0, i, 0)),
        pl.BlockSpec((_R, D), lambda i: (i, 0)),
        pl.BlockSpec((_R, 1), lambda i: (i, 0)),
        pl.BlockSpec((1, D), lambda i: (0, 0)),
        pl.BlockSpec((1, D), lambda i: (0, 0)),
        pl.BlockSpec((1, D), lambda i: (0, 0)),
        pl.BlockSpec((D, D), lambda i: (0, 0)),
    ],
    out_specs=pl.BlockSpec((_R, D), lambda i: (i, 0)),
    out_shape=jax.ShapeDtypeStruct((N, D), jnp.float32),
)

_c_call = pl.pallas_call(
    _c_body,
    grid=(N // _R,),
    in_specs=[
        pl.BlockSpec((NC, _R, D), lambda i: (0, i, 0)),
        pl.BlockSpec((_R, D), lambda i: (i, 0)),
        pl.BlockSpec((_R, 1), lambda i: (i, 0)),
        pl.BlockSpec((1, D), lambda i: (0, 0)),
    ],
    out_specs=pl.BlockSpec((_R, D), lambda i: (i, 0)),
    out_shape=jax.ShapeDtypeStruct((N, D), jnp.float32),
)


def kernel(x, adj_t, W0, b0, g0, be0, W1, b1, g1, be1, W2, b2):
    src = adj_t[0].astype(jnp.int32)
    dst = adj_t[1].astype(jnp.int32)
    ones_deg = jnp.ones((CHUNK, DEGW), jnp.float32)
    zeros_deg = jnp.zeros((RPT, DEGW), jnp.float32)
    zeros_row = jnp.zeros((RPT, D), jnp.float32)
    b0r, g0r, be0r = b0.reshape(1, D), g0.reshape(1, D), be0.reshape(1, D)
    b1r, g1r, be1r = b1.reshape(1, D), g1.reshape(1, D), be1.reshape(1, D)
    b2r = b2.reshape(1, D)

    degp = _deg_sc(dst, ones_deg, zeros_deg)
    y0, dinv = _a_call(degp, x, W0)
    p0 = _spmm_sc(y0, src, dst, zeros_row)
    y1 = _b_call(p0, y0, dinv, b0r, g0r, be0r, W1)
    p1 = _spmm_sc(y1, src, dst, zeros_row)
    y2 = _b_call(p1, y1, dinv, b1r, g1r, be1r, W2)
    p2 = _spmm_sc(y2, src, dst, zeros_row)
    return _c_call(p2, y2, dinv, b2r)


# speedup vs baseline: 11.1731x; 11.1731x over previous
import functools

import jax
import jax.numpy as jnp
from jax import lax
from jax.experimental import pallas as pl
from jax.experimental.pallas import tpu as pltpu
from jax.experimental.pallas import tpu_sc as plsc

N = 10000
D = 128
E = 320000

NC = 2
NS = 16
NW = NC * NS
EPW = E // NW
CHUNK = 80
STEPS = EPW // CHUNK
NPAD = 10240
RPT = NPAD // NS
DEGW = 16

_BN_SCALE = 1.0 / (1.0 + 1e-5) ** 0.5

_sc_mesh = plsc.VectorSubcoreMesh(
    core_axis_name="c", subcore_axis_name="s", num_cores=NC, num_subcores=NS)


@functools.partial(
    pl.kernel,
    out_type=jax.ShapeDtypeStruct((NC, NPAD), jnp.float32),
    mesh=_sc_mesh,
    scratch_types=[
        pltpu.VMEM((CHUNK,), jnp.int32),
        pltpu.VMEM((CHUNK,), jnp.float32),
        pltpu.VMEM_SHARED((NPAD,), jnp.float32),
    ],
)
def _deg_sc(dst_hbm, zeros_hbm, out_hbm, didx, ones_v, acc):
    c = lax.axis_index("c")
    s = lax.axis_index("s")
    w = s * NC + c

    def fill(i, carry):
        ones_v[pl.ds(i * 16, 16)] = jnp.ones((16,), jnp.float32)
        return carry

    lax.fori_loop(0, CHUNK // 16, fill, 0)
    pltpu.sync_copy(zeros_hbm, acc.at[pl.ds(s * RPT, RPT)])
    plsc.subcore_barrier()

    def step(j, carry):
        base = w * EPW + j * CHUNK
        pltpu.sync_copy(dst_hbm.at[pl.ds(base, CHUNK)], didx)
        pltpu.sync_copy(ones_v, acc.at[didx], add=True)
        return carry

    lax.fori_loop(0, STEPS, step, 0)
    plsc.subcore_barrier()
    pltpu.sync_copy(acc.at[pl.ds(s * RPT, RPT)], out_hbm.at[c, pl.ds(s * RPT, RPT)])


@functools.partial(
    pl.kernel,
    out_type=jax.ShapeDtypeStruct((NC, NPAD, D), jnp.float32),
    mesh=_sc_mesh,
    scratch_types=[
        pltpu.VMEM((CHUNK,), jnp.int32),
        pltpu.VMEM((CHUNK,), jnp.int32),
        pltpu.VMEM((CHUNK, D), jnp.float32),
        pltpu.SemaphoreType.DMA,
        pltpu.VMEM_SHARED((NPAD, D), jnp.float32),
    ],
)
def _spmm_sc(y_hbm, src_hbm, dst_hbm, zeros_hbm, out_hbm,
             sidx, didx, rows, sem, acc):
    c = lax.axis_index("c")
    s = lax.axis_index("s")
    w = s * NC + c
    pltpu.sync_copy(zeros_hbm, acc.at[pl.ds(s * RPT, RPT)])
    plsc.subcore_barrier()

    def step(j, carry):
        base = w * EPW + j * CHUNK
        pltpu.sync_copy(src_hbm.at[pl.ds(base, CHUNK)], sidx)
        pltpu.sync_copy(dst_hbm.at[pl.ds(base, CHUNK)], didx)
        pltpu.async_copy(y_hbm.at[sidx], rows, sem).wait()
        pltpu.sync_copy(rows, acc.at[didx], add=True)
        return carry

    lax.fori_loop(0, STEPS, step, 0)
    plsc.subcore_barrier()
    pltpu.sync_copy(acc.at[pl.ds(s * RPT, RPT)], out_hbm.at[c, pl.ds(s * RPT, RPT)])


_R = 1000


def _a_body(degp_ref, x_ref, w_ref, y_ref, dinv_ref):
    deg = 1.0 + degp_ref[0] + degp_ref[1]
    dinv = lax.rsqrt(deg)
    dinv_ref[...] = dinv
    y_ref[...] = jnp.dot(x_ref[...], w_ref[...],
                         preferred_element_type=jnp.float32) * dinv


def _b_body(p_ref, y_ref, dinv_ref, b_ref, g_ref, be_ref, w_ref, yn_ref):
    dinv = dinv_ref[...]
    z = (p_ref[0] + p_ref[1] + y_ref[...]) * dinv + b_ref[...]
    t = jnp.maximum(z * (g_ref[...] * _BN_SCALE) + be_ref[...], 0.0)
    yn_ref[...] = jnp.dot(t, w_ref[...],
                          preferred_element_type=jnp.float32) * dinv


def _c_body(p_ref, y_ref, dinv_ref, b_ref, o_ref):
    z = (p_ref[0] + p_ref[1] + y_ref[...]) * dinv_ref[...] + b_ref[...]
    m = jnp.max(z, axis=1, keepdims=True)
    lse = jnp.log(jnp.sum(jnp.exp(z - m), axis=1, keepdims=True)) + m
    o_ref[...] = z - lse


_a_call = pl.pallas_call(
    _a_body,
    grid=(N // _R,),
    in_specs=[
        pl.BlockSpec((NC, _R, 1), lambda i: (0, i, 0)),
        pl.BlockSpec((_R, D), lambda i: (i, 0)),
        pl.BlockSpec((D, D), lambda i: (0, 0)),
    ],
    out_specs=[
        pl.BlockSpec((_R, D), lambda i: (i, 0)),
        pl.BlockSpec((_R, 1), lambda i: (i, 0)),
    ],
    out_shape=[
        jax.ShapeDtypeStruct((N, D), jnp.float32),
        jax.ShapeDtypeStruct((N, 1), jnp.float32),
    ],
)

_b_call = pl.pallas_call(
    _b_body,
    grid=(N // _R,),
    in_specs=[
        pl.BlockSpec((NC, _R, D), lambda i: (0, i, 0)),
        pl.BlockSpec((_R, D), lambda i: (i, 0)),
        pl.BlockSpec((_R, 1), lambda i: (i, 0)),
        pl.BlockSpec((1, D), lambda i: (0, 0)),
        pl.BlockSpec((1, D), lambda i: (0, 0)),
        pl.BlockSpec((1, D), lambda i: (0, 0)),
        pl.BlockSpec((D, D), lambda i: (0, 0)),
    ],
    out_specs=pl.BlockSpec((_R, D), lambda i: (i, 0)),
    out_shape=jax.ShapeDtypeStruct((N, D), jnp.float32),
)

_c_call = pl.pallas_call(
    _c_body,
    grid=(N // _R,),
    in_specs=[
        pl.BlockSpec((NC, _R, D), lambda i: (0, i, 0)),
        pl.BlockSpec((_R, D), lambda i: (i, 0)),
        pl.BlockSpec((_R, 1), lambda i: (i, 0)),
        pl.BlockSpec((1, D), lambda i: (0, 0)),
    ],
    out_specs=pl.BlockSpec((_R, D), lambda i: (i, 0)),
    out_shape=jax.ShapeDtypeStruct((N, D), jnp.float32),
)


def kernel(x, adj_t, W0, b0, g0, be0, W1, b1, g1, be1, W2, b2):
    src = adj_t[0].astype(jnp.int32)
    dst = adj_t[1].astype(jnp.int32)
    zeros_deg = jnp.zeros((RPT,), jnp.float32)
    zeros_row = jnp.zeros((RPT, D), jnp.float32)
    b0r, g0r, be0r = b0.reshape(1, D), g0.reshape(1, D), be0.reshape(1, D)
    b1r, g1r, be1r = b1.reshape(1, D), g1.reshape(1, D), be1.reshape(1, D)
    b2r = b2.reshape(1, D)

    degp = _deg_sc(dst, zeros_deg).reshape(NC, NPAD, 1)
    y0, dinv = _a_call(degp, x, W0)
    p0 = _spmm_sc(y0, src, dst, zeros_row)
    y1 = _b_call(p0, y0, dinv, b0r, g0r, be0r, W1)
    p1 = _spmm_sc(y1, src, dst, zeros_row)
    y2 = _b_call(p1, y1, dinv, b1r, g1r, be1r, W2)
    p2 = _spmm_sc(y2, src, dst, zeros_row)
    return _c_call(p2, y2, dinv, b2r)


# speedup vs baseline: 23.8120x; 2.1312x over previous
import functools

import jax
import jax.numpy as jnp
from jax import lax
from jax.experimental import pallas as pl
from jax.experimental.pallas import tpu as pltpu
from jax.experimental.pallas import tpu_sc as plsc

N = 10000
D = 128
E = 320000

NC = 2
NS = 16
NW = NC * NS
EPW = E // NW
CHUNK = 80
STEPS = EPW // CHUNK
NPAD = 10240
RPT = NPAD // NS
DEGW = 16

_BN_SCALE = 1.0 / (1.0 + 1e-5) ** 0.5

_sc_mesh = plsc.VectorSubcoreMesh(
    core_axis_name="c", subcore_axis_name="s", num_cores=NC, num_subcores=NS)


@functools.partial(
    pl.kernel,
    out_type=jax.ShapeDtypeStruct((NC, NPAD), jnp.float32),
    mesh=_sc_mesh,
    scratch_types=[
        pltpu.VMEM((CHUNK,), jnp.int32),
        pltpu.VMEM((CHUNK,), jnp.float32),
        pltpu.VMEM_SHARED((NPAD,), jnp.float32),
    ],
)
def _deg_sc(dst_hbm, zeros_hbm, out_hbm, didx, ones_v, acc):
    c = lax.axis_index("c")
    s = lax.axis_index("s")
    w = s * NC + c

    def fill(i, carry):
        ones_v[pl.ds(i * 16, 16)] = jnp.ones((16,), jnp.float32)
        return carry

    lax.fori_loop(0, CHUNK // 16, fill, 0)
    pltpu.sync_copy(zeros_hbm, acc.at[pl.ds(s * RPT, RPT)])
    plsc.subcore_barrier()

    def step(j, carry):
        base = w * EPW + j * CHUNK
        pltpu.sync_copy(dst_hbm.at[pl.ds(base, CHUNK)], didx)
        pltpu.sync_copy(ones_v, acc.at[didx], add=True)
        return carry

    lax.fori_loop(0, STEPS, step, 0)
    plsc.subcore_barrier()
    pltpu.sync_copy(acc.at[pl.ds(s * RPT, RPT)], out_hbm.at[c, pl.ds(s * RPT, RPT)])


NBUF = 2


@functools.partial(
    pl.kernel,
    out_type=jax.ShapeDtypeStruct((NC, NPAD, D), jnp.float32),
    mesh=_sc_mesh,
    scratch_types=[
        pltpu.VMEM((EPW,), jnp.int32),
        pltpu.VMEM((STEPS, CHUNK), jnp.int32),
        pltpu.VMEM((NBUF, CHUNK, D), jnp.float32),
        pltpu.SemaphoreType.DMA,
        pltpu.SemaphoreType.DMA,
        pltpu.VMEM_SHARED((NPAD, D), jnp.float32),
    ],
)
def _spmm_sc(y_hbm, src_hbm, dst_hbm, zeros_hbm, out_hbm,
             sidx, didx, rows, sem0, sem1, acc):
    c = lax.axis_index("c")
    s = lax.axis_index("s")
    w = s * NC + c
    sems = (sem0, sem1)
    pltpu.async_copy(src_hbm.at[pl.ds(w * EPW, EPW)], sidx, sem0)
    pltpu.async_copy(dst_hbm.at[w], didx, sem1)
    pltpu.sync_copy(zeros_hbm, acc.at[pl.ds(s * RPT, RPT)])
    pltpu.make_async_copy(src_hbm.at[pl.ds(w * EPW, EPW)], sidx, sem0).wait()
    pltpu.make_async_copy(dst_hbm.at[w], didx, sem1).wait()
    plsc.subcore_barrier()

    def sidx_at(j):
        return sidx.at[pl.ds(j * CHUNK, CHUNK)]

    pltpu.async_copy(y_hbm.at[sidx_at(0)], rows.at[0], sem0)
    pltpu.async_copy(y_hbm.at[sidx_at(1)], rows.at[1], sem1)

    def outer(g, carry):
        for b in range(NBUF):
            j = g * NBUF + b
            pltpu.make_async_copy(y_hbm.at[sidx_at(j)], rows.at[b],
                                  sems[b]).wait()
            pltpu.sync_copy(rows.at[b], acc.at[didx.at[j]], add=True)

            @pl.when(j + NBUF < STEPS)
            def _():
                pltpu.async_copy(y_hbm.at[sidx_at(j + NBUF)], rows.at[b],
                                 sems[b])
        return carry

    lax.fori_loop(0, STEPS // NBUF, outer, 0)
    for j in range((STEPS // NBUF) * NBUF, STEPS):
        b = j % NBUF
        pltpu.make_async_copy(y_hbm.at[sidx_at(j)], rows.at[b],
                              sems[b]).wait()
        pltpu.sync_copy(rows.at[b], acc.at[didx.at[j]], add=True)
    plsc.subcore_barrier()
    pltpu.sync_copy(acc.at[pl.ds(s * RPT, RPT)], out_hbm.at[c, pl.ds(s * RPT, RPT)])


_R = 1000


def _a_body(degp_ref, x_ref, w_ref, y_ref, dinv_ref):
    deg = 1.0 + degp_ref[0] + degp_ref[1]
    dinv = lax.rsqrt(deg)
    dinv_ref[...] = dinv
    y_ref[...] = jnp.dot(x_ref[...], w_ref[...],
                         preferred_element_type=jnp.float32) * dinv


def _b_body(p_ref, y_ref, dinv_ref, b_ref, g_ref, be_ref, w_ref, yn_ref):
    dinv = dinv_ref[...]
    z = (p_ref[0] + p_ref[1] + y_ref[...]) * dinv + b_ref[...]
    t = jnp.maximum(z * (g_ref[...] * _BN_SCALE) + be_ref[...], 0.0)
    yn_ref[...] = jnp.dot(t, w_ref[...],
                          preferred_element_type=jnp.float32) * dinv


def _c_body(p_ref, y_ref, dinv_ref, b_ref, o_ref):
    z = (p_ref[0] + p_ref[1] + y_ref[...]) * dinv_ref[...] + b_ref[...]
    m = jnp.max(z, axis=1, keepdims=True)
    lse = jnp.log(jnp.sum(jnp.exp(z - m), axis=1, keepdims=True)) + m
    o_ref[...] = z - lse


_a_call = pl.pallas_call(
    _a_body,
    grid=(N // _R,),
    in_specs=[
        pl.BlockSpec((NC, _R, 1), lambda i: (0, i, 0)),
        pl.BlockSpec((_R, D), lambda i: (i, 0)),
        pl.BlockSpec((D, D), lambda i: (0, 0)),
    ],
    out_specs=[
        pl.BlockSpec((_R, D), lambda i: (i, 0)),
        pl.BlockSpec((_R, 1), lambda i: (i, 0)),
    ],
    out_shape=[
        jax.ShapeDtypeStruct((N, D), jnp.float32),
        jax.ShapeDtypeStruct((N, 1), jnp.float32),
    ],
)

_b_call = pl.pallas_call(
    _b_body,
    grid=(N // _R,),
    in_specs=[
        pl.BlockSpec((NC, _R, D), lambda i: (0, i, 0)),
        pl.BlockSpec((_R, D), lambda i: (i, 0)),
        pl.BlockSpec((_R, 1), lambda i: (i, 0)),
        pl.BlockSpec((1, D), lambda i: (0, 0)),
        pl.BlockSpec((1, D), lambda i: (0, 0)),
        pl.BlockSpec((1, D), lambda i: (0, 0)),
        pl.BlockSpec((D, D), lambda i: (0, 0)),
    ],
    out_specs=pl.BlockSpec((_R, D), lambda i: (i, 0)),
    out_shape=jax.ShapeDtypeStruct((N, D), jnp.float32),
)

_c_call = pl.pallas_call(
    _c_body,
    grid=(N // _R,),
    in_specs=[
        pl.BlockSpec((NC, _R, D), lambda i: (0, i, 0)),
        pl.BlockSpec((_R, D), lambda i: (i, 0)),
        pl.BlockSpec((_R, 1), lambda i: (i, 0)),
        pl.BlockSpec((1, D), lambda i: (0, 0)),
    ],
    out_specs=pl.BlockSpec((_R, D), lambda i: (i, 0)),
    out_shape=jax.ShapeDtypeStruct((N, D), jnp.float32),
)


def kernel(x, adj_t, W0, b0, g0, be0, W1, b1, g1, be1, W2, b2):
    src = adj_t[0].astype(jnp.int32)
    dst = adj_t[1].astype(jnp.int32)
    zeros_deg = jnp.zeros((RPT,), jnp.float32)
    zeros_row = jnp.zeros((RPT, D), jnp.float32)
    b0r, g0r, be0r = b0.reshape(1, D), g0.reshape(1, D), be0.reshape(1, D)
    b1r, g1r, be1r = b1.reshape(1, D), g1.reshape(1, D), be1.reshape(1, D)
    b2r = b2.reshape(1, D)

    dst_t = dst.reshape(NW, STEPS, CHUNK)
    degp = _deg_sc(dst, zeros_deg).reshape(NC, NPAD, 1)
    y0, dinv = _a_call(degp, x, W0)
    p0 = _spmm_sc(y0, src, dst_t, zeros_row)
    y1 = _b_call(p0, y0, dinv, b0r, g0r, be0r, W1)
    p1 = _spmm_sc(y1, src, dst_t, zeros_row)
    y2 = _b_call(p1, y1, dinv, b1r, g1r, be1r, W2)
    p2 = _spmm_sc(y2, src, dst_t, zeros_row)
    return _c_call(p2, y2, dinv, b2r)


# speedup vs baseline: 30.0805x; 1.2633x over previous
import functools

import jax
import jax.numpy as jnp
from jax import lax
from jax.experimental import pallas as pl
from jax.experimental.pallas import tpu as pltpu
from jax.experimental.pallas import tpu_sc as plsc

N = 10000
D = 128
E = 320000

NC = 2
NS = 16
NW = NC * NS
EPW = E // NW
CHUNK = 80
STEPS = EPW // CHUNK
NPAD = 10240
RPT = NPAD // NS
DEGW = 16

_BN_SCALE = 1.0 / (1.0 + 1e-5) ** 0.5

_sc_mesh = plsc.VectorSubcoreMesh(
    core_axis_name="c", subcore_axis_name="s", num_cores=NC, num_subcores=NS)


_DEG_GRP = 5


@functools.partial(
    pl.kernel,
    out_type=jax.ShapeDtypeStruct((NC, NPAD), jnp.float32),
    mesh=_sc_mesh,
    scratch_types=[
        pltpu.VMEM((STEPS, CHUNK), jnp.int32),
        pltpu.VMEM((CHUNK,), jnp.float32),
        pltpu.SemaphoreType.DMA,
        pltpu.SemaphoreType.DMA,
        pltpu.VMEM_SHARED((NPAD,), jnp.float32),
    ],
)
def _deg_sc(dst_hbm, zeros_hbm, out_hbm, didx, ones_v, isem, ssem, acc):
    c = lax.axis_index("c")
    s = lax.axis_index("s")
    w = s * NC + c

    pltpu.async_copy(dst_hbm.at[w], didx, isem)

    def fill(i, carry):
        ones_v[pl.ds(i * 16, 16)] = jnp.ones((16,), jnp.float32)
        return carry

    lax.fori_loop(0, CHUNK // 16, fill, 0)
    pltpu.sync_copy(zeros_hbm, acc.at[pl.ds(s * RPT, RPT)])
    pltpu.make_async_copy(dst_hbm.at[w], didx, isem).wait()
    plsc.subcore_barrier()

    def group(g, carry):
        for b in range(_DEG_GRP):
            pltpu.async_copy(ones_v, acc.at[didx.at[g * _DEG_GRP + b]], ssem,
                             add=True)
        for b in range(_DEG_GRP):
            pltpu.make_async_copy(ones_v, acc.at[didx.at[0]], ssem).wait()
        return carry

    lax.fori_loop(0, STEPS // _DEG_GRP, group, 0)
    plsc.subcore_barrier()
    pltpu.sync_copy(acc.at[pl.ds(s * RPT, RPT)], out_hbm.at[c, pl.ds(s * RPT, RPT)])


NBUF = 3


@functools.partial(
    pl.kernel,
    out_type=jax.ShapeDtypeStruct((NC, NPAD, D), jnp.float32),
    mesh=_sc_mesh,
    scratch_types=[
        pltpu.VMEM((EPW,), jnp.int32),
        pltpu.VMEM((NBUF, CHUNK), jnp.int32),
        pltpu.VMEM((NBUF, CHUNK, D), jnp.float32),
        pltpu.SemaphoreType.DMA,
        pltpu.SemaphoreType.DMA,
        pltpu.SemaphoreType.DMA,
        pltpu.SemaphoreType.DMA,
        pltpu.SemaphoreType.DMA,
        pltpu.SemaphoreType.DMA,
        pltpu.SemaphoreType.DMA,
        pltpu.SemaphoreType.DMA,
        pltpu.SemaphoreType.DMA,
        pltpu.VMEM_SHARED((NPAD, D), jnp.float32),
    ],
)
def _spmm_sc(y_hbm, src_hbm, dst_hbm, zeros_hbm, out_hbm,
             sidx, dring, rows, g0, g1, g2, s0, s1, s2, d0, d1, d2, acc):
    c = lax.axis_index("c")
    s = lax.axis_index("s")
    w = s * NC + c
    gsems = (g0, g1, g2)
    ssems = (s0, s1, s2)
    dsems = (d0, d1, d2)
    pltpu.async_copy(src_hbm.at[pl.ds(w * EPW, EPW)], sidx, g0)
    pltpu.sync_copy(zeros_hbm, acc.at[pl.ds(s * RPT, RPT)])
    pltpu.make_async_copy(src_hbm.at[pl.ds(w * EPW, EPW)], sidx, g0).wait()
    plsc.subcore_barrier()

    def sidx_at(j):
        return sidx.at[pl.ds(j * CHUNK, CHUNK)]

    def didx_src(j):
        return dst_hbm.at[pl.ds(w * EPW + j * CHUNK, CHUNK)]

    def start_fetch(j, b):
        pltpu.async_copy(didx_src(j), dring.at[b], dsems[b])
        pltpu.async_copy(y_hbm.at[sidx_at(j)], rows.at[b], gsems[b])

    def wait_gather(j, b):
        pltpu.make_async_copy(y_hbm.at[sidx_at(j)], rows.at[b],
                              gsems[b]).wait()
        pltpu.make_async_copy(didx_src(j), dring.at[b], dsems[b]).wait()

    def start_scatter(b):
        pltpu.async_copy(rows.at[b], acc.at[dring.at[b]], ssems[b], add=True)

    def wait_scatter(b):
        pltpu.make_async_copy(rows.at[b], acc.at[dring.at[b]],
                              ssems[b]).wait()

    start_fetch(0, 0)
    start_fetch(1, 1)

    def outer(g, carry):
        for k in range(NBUF):
            j = g * NBUF + k
            bp = (k + NBUF - 1) % NBUF
            wait_gather(j, k)
            start_scatter(k)

            @pl.when(j >= 1)
            def _():
                wait_scatter(bp)

            start_fetch(j + 2, bp)
        return carry

    lax.fori_loop(0, STEPS // NBUF, outer, 0)
    for j in range((STEPS // NBUF) * NBUF, STEPS):
        k = j % NBUF
        bp = (k + NBUF - 1) % NBUF
        wait_gather(j, k)
        start_scatter(k)
        wait_scatter(bp)
    wait_scatter((STEPS - 1) % NBUF)
    plsc.subcore_barrier()
    pltpu.sync_copy(acc.at[pl.ds(s * RPT, RPT)], out_hbm.at[c, pl.ds(s * RPT, RPT)])


_R = 1000


def _a_body(degp_ref, x_ref, w_ref, y_ref, dinv_ref):
    deg = 1.0 + degp_ref[0] + degp_ref[1]
    dinv = lax.rsqrt(deg)
    dinv_ref[...] = dinv
    y_ref[...] = jnp.dot(x_ref[...], w_ref[...],
                         preferred_element_type=jnp.float32) * dinv


def _b_body(p_ref, y_ref, dinv_ref, b_ref, g_ref, be_ref, w_ref, yn_ref):
    dinv = dinv_ref[...]
    z = (p_ref[0] + p_ref[1] + y_ref[...]) * dinv + b_ref[...]
    t = jnp.maximum(z * (g_ref[...] * _BN_SCALE) + be_ref[...], 0.0)
    yn_ref[...] = jnp.dot(t, w_ref[...],
                          preferred_element_type=jnp.float32) * dinv


def _c_body(p_ref, y_ref, dinv_ref, b_ref, o_ref):
    z = (p_ref[0] + p_ref[1] + y_ref[...]) * dinv_ref[...] + b_ref[...]
    m = jnp.max(z, axis=1, keepdims=True)
    lse = jnp.log(jnp.sum(jnp.exp(z - m), axis=1, keepdims=True)) + m
    o_ref[...] = z - lse


_a_call = pl.pallas_call(
    _a_body,
    grid=(N // _R,),
    in_specs=[
        pl.BlockSpec((NC, _R, 1), lambda i: (0, i, 0)),
        pl.BlockSpec((_R, D), lambda i: (i, 0)),
        pl.BlockSpec((D, D), lambda i: (0, 0)),
    ],
    out_specs=[
        pl.BlockSpec((_R, D), lambda i: (i, 0)),
        pl.BlockSpec((_R, 1), lambda i: (i, 0)),
    ],
    out_shape=[
        jax.ShapeDtypeStruct((N, D), jnp.float32),
        jax.ShapeDtypeStruct((N, 1), jnp.float32),
    ],
)

_b_call = pl.pallas_call(
    _b_body,
    grid=(N // _R,),
    in_specs=[
        pl.BlockSpec((NC, _R, D), lambda i: (0, i, 0)),
        pl.BlockSpec((_R, D), lambda i: (i, 0)),
        pl.BlockSpec((_R, 1), lambda i: (i, 0)),
        pl.BlockSpec((1, D), lambda i: (0, 0)),
        pl.BlockSpec((1, D), lambda i: (0, 0)),
        pl.BlockSpec((1, D), lambda i: (0, 0)),
        pl.BlockSpec((D, D), lambda i: (0, 0)),
    ],
    out_specs=pl.BlockSpec((_R, D), lambda i: (i, 0)),
    out_shape=jax.ShapeDtypeStruct((N, D), jnp.float32),
)

_c_call = pl.pallas_call(
    _c_body,
    grid=(N // _R,),
    in_specs=[
        pl.BlockSpec((NC, _R, D), lambda i: (0, i, 0)),
        pl.BlockSpec((_R, D), lambda i: (i, 0)),
        pl.BlockSpec((_R, 1), lambda i: (i, 0)),
        pl.BlockSpec((1, D), lambda i: (0, 0)),
    ],
    out_specs=pl.BlockSpec((_R, D), lambda i: (i, 0)),
    out_shape=jax.ShapeDtypeStruct((N, D), jnp.float32),
)


def kernel(x, adj_t, W0, b0, g0, be0, W1, b1, g1, be1, W2, b2):
    src = adj_t[0].astype(jnp.int32)
    dst = adj_t[1].astype(jnp.int32)
    zeros_deg = jnp.zeros((RPT,), jnp.float32)
    zeros_row = jnp.zeros((RPT, D), jnp.float32)
    b0r, g0r, be0r = b0.reshape(1, D), g0.reshape(1, D), be0.reshape(1, D)
    b1r, g1r, be1r = b1.reshape(1, D), g1.reshape(1, D), be1.reshape(1, D)
    b2r = b2.reshape(1, D)

    dst_t = dst.reshape(NW, STEPS, CHUNK)
    degp = _deg_sc(dst_t, zeros_deg).reshape(NC, NPAD, 1)
    y0, dinv = _a_call(degp, x, W0)
    p0 = _spmm_sc(y0, src, dst, zeros_row)
    y1 = _b_call(p0, y0, dinv, b0r, g0r, be0r, W1)
    p1 = _spmm_sc(y1, src, dst, zeros_row)
    y2 = _b_call(p1, y1, dinv, b1r, g1r, be1r, W2)
    p2 = _spmm_sc(y2, src, dst, zeros_row)
    return _c_call(p2, y2, dinv, b2r)


# speedup vs baseline: 30.3648x; 1.0095x over previous
import functools

import jax
import jax.numpy as jnp
from jax import lax
from jax.experimental import pallas as pl
from jax.experimental.pallas import tpu as pltpu
from jax.experimental.pallas import tpu_sc as plsc

N = 10000
D = 128
E = 320000

NC = 2
NS = 16
NW = NC * NS
EPW = E // NW
CHUNK = 80
STEPS = EPW // CHUNK
NPAD = 10240
RPT = NPAD // NS
DEGW = 16

_BN_SCALE = 1.0 / (1.0 + 1e-5) ** 0.5

_sc_mesh = plsc.VectorSubcoreMesh(
    core_axis_name="c", subcore_axis_name="s", num_cores=NC, num_subcores=NS)


_DEG_GRP = 5


@functools.partial(
    pl.kernel,
    out_type=jax.ShapeDtypeStruct((NC, NPAD), jnp.float32),
    mesh=_sc_mesh,
    scratch_types=[
        pltpu.VMEM((STEPS, CHUNK), jnp.int32),
        pltpu.VMEM((CHUNK,), jnp.float32),
        pltpu.SemaphoreType.DMA,
        pltpu.SemaphoreType.DMA,
        pltpu.VMEM_SHARED((NPAD,), jnp.float32),
    ],
)
def _deg_sc(dst_hbm, zeros_hbm, out_hbm, didx, ones_v, isem, ssem, acc):
    c = lax.axis_index("c")
    s = lax.axis_index("s")
    w = s * NC + c

    pltpu.async_copy(dst_hbm.at[w], didx, isem)

    def fill(i, carry):
        ones_v[pl.ds(i * 16, 16)] = jnp.ones((16,), jnp.float32)
        return carry

    lax.fori_loop(0, CHUNK // 16, fill, 0)
    pltpu.sync_copy(zeros_hbm, acc.at[pl.ds(s * RPT, RPT)])
    pltpu.make_async_copy(dst_hbm.at[w], didx, isem).wait()
    plsc.subcore_barrier()

    def group(g, carry):
        for b in range(_DEG_GRP):
            pltpu.async_copy(ones_v, acc.at[didx.at[g * _DEG_GRP + b]], ssem,
                             add=True)
        for b in range(_DEG_GRP):
            pltpu.make_async_copy(ones_v, acc.at[didx.at[0]], ssem).wait()
        return carry

    lax.fori_loop(0, STEPS // _DEG_GRP, group, 0)
    plsc.subcore_barrier()
    pltpu.sync_copy(acc.at[pl.ds(s * RPT, RPT)], out_hbm.at[c, pl.ds(s * RPT, RPT)])


NBUF = 3


@functools.partial(
    pl.kernel,
    out_type=jax.ShapeDtypeStruct((NC, NPAD, D), jnp.float32),
    mesh=_sc_mesh,
    scratch_types=[
        pltpu.VMEM((EPW,), jnp.int32),
        pltpu.VMEM((NBUF, CHUNK), jnp.int32),
        pltpu.VMEM((NBUF, CHUNK, D), jnp.float32),
        pltpu.SemaphoreType.DMA,
        pltpu.SemaphoreType.DMA,
        pltpu.SemaphoreType.DMA,
        pltpu.SemaphoreType.DMA,
        pltpu.SemaphoreType.DMA,
        pltpu.SemaphoreType.DMA,
        pltpu.SemaphoreType.DMA,
        pltpu.SemaphoreType.DMA,
        pltpu.SemaphoreType.DMA,
        pltpu.VMEM_SHARED((NPAD, D), jnp.float32),
    ],
)
def _spmm_sc(y_hbm, src_hbm, dst_hbm, zeros_hbm, out_hbm,
             sidx, dring, rows, g0, g1, g2, s0, s1, s2, d0, d1, d2, acc):
    c = lax.axis_index("c")
    s = lax.axis_index("s")
    w = s * NC + c
    gsems = (g0, g1, g2)
    ssems = (s0, s1, s2)
    dsems = (d0, d1, d2)
    pltpu.async_copy(src_hbm.at[pl.ds(w * EPW, EPW)], sidx, g0)
    init_src = y_hbm.at[pl.ds(s * RPT, RPT)]
    zero_src = zeros_hbm.at[pl.ds(0, RPT)]
    acc_dst = acc.at[pl.ds(s * RPT, RPT)]

    @pl.when(c == 0)
    def _():
        pltpu.async_copy(init_src, acc_dst, s0)

    @pl.when(c != 0)
    def _():
        pltpu.async_copy(zero_src, acc_dst, s0)

    pltpu.make_async_copy(src_hbm.at[pl.ds(w * EPW, EPW)], sidx, g0).wait()
    pltpu.make_async_copy(init_src, acc_dst, s0).wait()
    plsc.subcore_barrier()

    def sidx_at(j):
        return sidx.at[pl.ds(j * CHUNK, CHUNK)]

    def didx_src(j):
        return dst_hbm.at[pl.ds(w * EPW + j * CHUNK, CHUNK)]

    def start_fetch(j, b):
        pltpu.async_copy(didx_src(j), dring.at[b], dsems[b])
        pltpu.async_copy(y_hbm.at[sidx_at(j)], rows.at[b], gsems[b])

    def wait_gather(j, b):
        pltpu.make_async_copy(y_hbm.at[sidx_at(j)], rows.at[b],
                              gsems[b]).wait()
        pltpu.make_async_copy(didx_src(j), dring.at[b], dsems[b]).wait()

    def start_scatter(b):
        pltpu.async_copy(rows.at[b], acc.at[dring.at[b]], ssems[b], add=True)

    def wait_scatter(b):
        pltpu.make_async_copy(rows.at[b], acc.at[dring.at[b]],
                              ssems[b]).wait()

    start_fetch(0, 0)
    start_fetch(1, 1)

    def outer(g, carry):
        for k in range(NBUF):
            j = g * NBUF + k
            bp = (k + NBUF - 1) % NBUF
            wait_gather(j, k)
            start_scatter(k)

            @pl.when(j >= 1)
            def _():
                wait_scatter(bp)

            start_fetch(j + 2, bp)
        return carry

    lax.fori_loop(0, STEPS // NBUF, outer, 0)
    for j in range((STEPS // NBUF) * NBUF, STEPS):
        k = j % NBUF
        bp = (k + NBUF - 1) % NBUF
        wait_gather(j, k)
        start_scatter(k)
        wait_scatter(bp)
    wait_scatter((STEPS - 1) % NBUF)
    plsc.subcore_barrier()
    pltpu.sync_copy(acc.at[pl.ds(s * RPT, RPT)], out_hbm.at[c, pl.ds(s * RPT, RPT)])


_R = 1024


def _a_body(degp_ref, x_ref, w_ref, y_ref, dinv_ref):
    deg = 1.0 + degp_ref[0] + degp_ref[1]
    dinv = lax.rsqrt(deg)
    dinv_ref[...] = dinv
    y_ref[...] = jnp.dot(x_ref[...], w_ref[...],
                         preferred_element_type=jnp.float32) * dinv


def _b_body(p_ref, dinv_ref, b_ref, g_ref, be_ref, w_ref, yn_ref):
    dinv = dinv_ref[...]
    z = (p_ref[0] + p_ref[1]) * dinv + b_ref[...]
    t = jnp.maximum(z * (g_ref[...] * _BN_SCALE) + be_ref[...], 0.0)
    yn_ref[...] = jnp.dot(t, w_ref[...],
                          preferred_element_type=jnp.float32) * dinv


def _c_body(p_ref, dinv_ref, b_ref, o_ref):
    z = (p_ref[0] + p_ref[1]) * dinv_ref[...] + b_ref[...]
    m = jnp.max(z, axis=1, keepdims=True)
    lse = jnp.log(jnp.sum(jnp.exp(z - m), axis=1, keepdims=True)) + m
    o_ref[...] = z - lse


_a_call = pl.pallas_call(
    _a_body,
    grid=(NPAD // _R,),
    in_specs=[
        pl.BlockSpec((NC, _R, 1), lambda i: (0, i, 0)),
        pl.BlockSpec((_R, D), lambda i: (i, 0)),
        pl.BlockSpec((D, D), lambda i: (0, 0)),
    ],
    out_specs=[
        pl.BlockSpec((_R, D), lambda i: (i, 0)),
        pl.BlockSpec((_R, 1), lambda i: (i, 0)),
    ],
    out_shape=[
        jax.ShapeDtypeStruct((NPAD, D), jnp.float32),
        jax.ShapeDtypeStruct((NPAD, 1), jnp.float32),
    ],
)

_b_call = pl.pallas_call(
    _b_body,
    grid=(NPAD // _R,),
    in_specs=[
        pl.BlockSpec((NC, _R, D), lambda i: (0, i, 0)),
        pl.BlockSpec((_R, 1), lambda i: (i, 0)),
        pl.BlockSpec((1, D), lambda i: (0, 0)),
        pl.BlockSpec((1, D), lambda i: (0, 0)),
        pl.BlockSpec((1, D), lambda i: (0, 0)),
        pl.BlockSpec((D, D), lambda i: (0, 0)),
    ],
    out_specs=pl.BlockSpec((_R, D), lambda i: (i, 0)),
    out_shape=jax.ShapeDtypeStruct((NPAD, D), jnp.float32),
)

_c_call = pl.pallas_call(
    _c_body,
    grid=(NPAD // _R,),
    in_specs=[
        pl.BlockSpec((NC, _R, D), lambda i: (0, i, 0)),
        pl.BlockSpec((_R, 1), lambda i: (i, 0)),
        pl.BlockSpec((1, D), lambda i: (0, 0)),
    ],
    out_specs=pl.BlockSpec((_R, D), lambda i: (i, 0)),
    out_shape=jax.ShapeDtypeStruct((NPAD, D), jnp.float32),
)


def kernel(x, adj_t, W0, b0, g0, be0, W1, b1, g1, be1, W2, b2):
    src = adj_t[0].astype(jnp.int32)
    dst = adj_t[1].astype(jnp.int32)
    zeros_deg = jnp.zeros((RPT,), jnp.float32)
    zeros_row = jnp.zeros((RPT, D), jnp.float32)
    b0r, g0r, be0r = b0.reshape(1, D), g0.reshape(1, D), be0.reshape(1, D)
    b1r, g1r, be1r = b1.reshape(1, D), g1.reshape(1, D), be1.reshape(1, D)
    b2r = b2.reshape(1, D)
    xp = jnp.pad(x, ((0, NPAD - N), (0, 0)))

    dst_t = dst.reshape(NW, STEPS, CHUNK)
    degp = _deg_sc(dst_t, zeros_deg).reshape(NC, NPAD, 1)
    y0, dinv = _a_call(degp, xp, W0)
    p0 = _spmm_sc(y0, src, dst, zeros_row)
    y1 = _b_call(p0, dinv, b0r, g0r, be0r, W1)
    p1 = _spmm_sc(y1, src, dst, zeros_row)
    y2 = _b_call(p1, dinv, b1r, g1r, be1r, W2)
    p2 = _spmm_sc(y2, src, dst, zeros_row)
    return _c_call(p2, dinv, b2r)[:N]


# speedup vs baseline: 31.6204x; 1.0414x over previous
import functools

import jax
import jax.numpy as jnp
from jax import lax
from jax.experimental import pallas as pl
from jax.experimental.pallas import tpu as pltpu
from jax.experimental.pallas import tpu_sc as plsc

N = 10000
D = 128
E = 320000

NC = 2
NS = 16
NW = NC * NS
EPW = E // NW
CHUNK = 80
STEPS = EPW // CHUNK
NPAD = 10240
RPT = NPAD // NS
DEGW = 16

_BN_SCALE = 1.0 / (1.0 + 1e-5) ** 0.5

_sc_mesh = plsc.VectorSubcoreMesh(
    core_axis_name="c", subcore_axis_name="s", num_cores=NC, num_subcores=NS)


_DEG_GRP = 5


@functools.partial(
    pl.kernel,
    out_type=jax.ShapeDtypeStruct((NC, NPAD), jnp.float32),
    mesh=_sc_mesh,
    scratch_types=[
        pltpu.VMEM((STEPS, CHUNK), jnp.int32),
        pltpu.VMEM((CHUNK,), jnp.float32),
        pltpu.SemaphoreType.DMA,
        pltpu.SemaphoreType.DMA,
        pltpu.VMEM_SHARED((NPAD,), jnp.float32),
    ],
)
def _deg_sc(dst_hbm, zeros_hbm, out_hbm, didx, ones_v, isem, ssem, acc):
    c = lax.axis_index("c")
    s = lax.axis_index("s")
    w = s * NC + c

    pltpu.async_copy(dst_hbm.at[w], didx, isem)

    def fill(i, carry):
        ones_v[pl.ds(i * 16, 16)] = jnp.ones((16,), jnp.float32)
        return carry

    lax.fori_loop(0, CHUNK // 16, fill, 0)
    pltpu.sync_copy(zeros_hbm, acc.at[pl.ds(s * RPT, RPT)])
    pltpu.make_async_copy(dst_hbm.at[w], didx, isem).wait()
    plsc.subcore_barrier()

    def group(g, carry):
        for b in range(_DEG_GRP):
            pltpu.async_copy(ones_v, acc.at[didx.at[g * _DEG_GRP + b]], ssem,
                             add=True)
        for b in range(_DEG_GRP):
            pltpu.make_async_copy(ones_v, acc.at[didx.at[0]], ssem).wait()
        return carry

    lax.fori_loop(0, STEPS // _DEG_GRP, group, 0)
    plsc.subcore_barrier()
    pltpu.sync_copy(acc.at[pl.ds(s * RPT, RPT)], out_hbm.at[c, pl.ds(s * RPT, RPT)])


NBUF = 3


@functools.partial(
    pl.kernel,
    out_type=jax.ShapeDtypeStruct((NC, NPAD, D), jnp.float32),
    mesh=_sc_mesh,
    scratch_types=[
        pltpu.VMEM((EPW,), jnp.int32),
        pltpu.VMEM((NBUF, CHUNK), jnp.int32),
        pltpu.VMEM((NBUF, CHUNK, D), jnp.float32),
        pltpu.SemaphoreType.DMA,
        pltpu.SemaphoreType.DMA,
        pltpu.SemaphoreType.DMA,
        pltpu.SemaphoreType.DMA,
        pltpu.SemaphoreType.DMA,
        pltpu.SemaphoreType.DMA,
        pltpu.SemaphoreType.DMA,
        pltpu.SemaphoreType.DMA,
        pltpu.SemaphoreType.DMA,
        pltpu.VMEM_SHARED((NPAD, D), jnp.float32),
    ],
)
def _spmm_sc(y_hbm, src_hbm, dst_hbm, zeros_hbm, out_hbm,
             sidx, dring, rows, g0, g1, g2, s0, s1, s2, d0, d1, d2, acc):
    c = lax.axis_index("c")
    s = lax.axis_index("s")
    w = s * NC + c
    gsems = (g0, g1, g2)
    ssems = (s0, s1, s2)
    dsems = (d0, d1, d2)
    pltpu.async_copy(src_hbm.at[pl.ds(w * EPW, EPW)], sidx, g0)
    init_src = y_hbm.at[pl.ds(s * RPT, RPT)]
    zero_src = zeros_hbm.at[pl.ds(0, RPT)]
    acc_dst = acc.at[pl.ds(s * RPT, RPT)]

    @pl.when(c == 0)
    def _():
        pltpu.async_copy(init_src, acc_dst, s0)

    @pl.when(c != 0)
    def _():
        pltpu.async_copy(zero_src, acc_dst, s0)

    pltpu.make_async_copy(src_hbm.at[pl.ds(w * EPW, EPW)], sidx, g0).wait()
    pltpu.make_async_copy(init_src, acc_dst, s0).wait()
    plsc.subcore_barrier()

    def sidx_at(j):
        return sidx.at[pl.ds(j * CHUNK, CHUNK)]

    def didx_src(j):
        return dst_hbm.at[pl.ds(w * EPW + j * CHUNK, CHUNK)]

    def start_fetch(j, b):
        pltpu.async_copy(didx_src(j), dring.at[b], dsems[b])
        pltpu.async_copy(y_hbm.at[sidx_at(j)], rows.at[b], gsems[b])

    def wait_gather(j, b):
        pltpu.make_async_copy(y_hbm.at[sidx_at(j)], rows.at[b],
                              gsems[b]).wait()
        pltpu.make_async_copy(didx_src(j), dring.at[b], dsems[b]).wait()

    def start_scatter(b):
        pltpu.async_copy(rows.at[b], acc.at[dring.at[b]], ssems[b], add=True)

    def wait_scatter(b):
        pltpu.make_async_copy(rows.at[b], acc.at[dring.at[b]],
                              ssems[b]).wait()

    start_fetch(0, 0)
    start_fetch(1, 1)

    def outer(g, carry):
        for k in range(NBUF):
            j = g * NBUF + k
            bp = (k + NBUF - 1) % NBUF
            wait_gather(j, k)
            start_scatter(k)

            @pl.when(j >= 1)
            def _():
                wait_scatter(bp)

            start_fetch(j + 2, bp)
        return carry

    lax.fori_loop(0, STEPS // NBUF, outer, 0)
    for j in range((STEPS // NBUF) * NBUF, STEPS):
        k = j % NBUF
        bp = (k + NBUF - 1) % NBUF
        wait_gather(j, k)
        start_scatter(k)
        wait_scatter(bp)
    wait_scatter((STEPS - 1) % NBUF)
    plsc.subcore_barrier()
    pltpu.sync_copy(acc.at[pl.ds(s * RPT, RPT)], out_hbm.at[c, pl.ds(s * RPT, RPT)])


_R = 2048


def _a_body(degp_ref, x_ref, w_ref, y_ref, dinv_ref):
    deg = 1.0 + degp_ref[0] + degp_ref[1]
    dinv = lax.rsqrt(deg)
    dinv_ref[...] = dinv
    y_ref[...] = jnp.dot(x_ref[...], w_ref[...],
                         preferred_element_type=jnp.float32) * dinv


def _b_body(p_ref, dinv_ref, b_ref, g_ref, be_ref, w_ref, yn_ref):
    dinv = dinv_ref[...]
    z = (p_ref[0] + p_ref[1]) * dinv + b_ref[...]
    t = jnp.maximum(z * (g_ref[...] * _BN_SCALE) + be_ref[...], 0.0)
    yn_ref[...] = jnp.dot(t, w_ref[...],
                          preferred_element_type=jnp.float32) * dinv


def _c_body(p_ref, dinv_ref, b_ref, o_ref):
    z = (p_ref[0] + p_ref[1]) * dinv_ref[...] + b_ref[...]
    m = jnp.max(z, axis=1, keepdims=True)
    lse = jnp.log(jnp.sum(jnp.exp(z - m), axis=1, keepdims=True)) + m
    o_ref[...] = z - lse


_a_call = pl.pallas_call(
    _a_body,
    grid=(NPAD // _R,),
    in_specs=[
        pl.BlockSpec((NC, _R, 1), lambda i: (0, i, 0)),
        pl.BlockSpec((_R, D), lambda i: (i, 0)),
        pl.BlockSpec((D, D), lambda i: (0, 0)),
    ],
    out_specs=[
        pl.BlockSpec((_R, D), lambda i: (i, 0)),
        pl.BlockSpec((_R, 1), lambda i: (i, 0)),
    ],
    out_shape=[
        jax.ShapeDtypeStruct((NPAD, D), jnp.float32),
        jax.ShapeDtypeStruct((NPAD, 1), jnp.float32),
    ],
)

_b_call = pl.pallas_call(
    _b_body,
    grid=(NPAD // _R,),
    in_specs=[
        pl.BlockSpec((NC, _R, D), lambda i: (0, i, 0)),
        pl.BlockSpec((_R, 1), lambda i: (i, 0)),
        pl.BlockSpec((1, D), lambda i: (0, 0)),
        pl.BlockSpec((1, D), lambda i: (0, 0)),
        pl.BlockSpec((1, D), lambda i: (0, 0)),
        pl.BlockSpec((D, D), lambda i: (0, 0)),
    ],
    out_specs=pl.BlockSpec((_R, D), lambda i: (i, 0)),
    out_shape=jax.ShapeDtypeStruct((NPAD, D), jnp.float32),
)

_c_call = pl.pallas_call(
    _c_body,
    grid=(NPAD // _R,),
    in_specs=[
        pl.BlockSpec((NC, _R, D), lambda i: (0, i, 0)),
        pl.BlockSpec((_R, 1), lambda i: (i, 0)),
        pl.BlockSpec((1, D), lambda i: (0, 0)),
    ],
    out_specs=pl.BlockSpec((_R, D), lambda i: (i, 0)),
    out_shape=jax.ShapeDtypeStruct((N, D), jnp.float32),
)


def kernel(x, adj_t, W0, b0, g0, be0, W1, b1, g1, be1, W2, b2):
    src = adj_t[0].astype(jnp.int32)
    dst = adj_t[1].astype(jnp.int32)
    zeros_deg = jnp.zeros((RPT,), jnp.float32)
    zeros_row = jnp.zeros((RPT, D), jnp.float32)
    b0r, g0r, be0r = b0.reshape(1, D), g0.reshape(1, D), be0.reshape(1, D)
    b1r, g1r, be1r = b1.reshape(1, D), g1.reshape(1, D), be1.reshape(1, D)
    b2r = b2.reshape(1, D)

    dst_t = dst.reshape(NW, STEPS, CHUNK)
    degp = _deg_sc(dst_t, zeros_deg).reshape(NC, NPAD, 1)
    y0, dinv = _a_call(degp, x, W0)
    p0 = _spmm_sc(y0, src, dst, zeros_row)
    y1 = _b_call(p0, dinv, b0r, g0r, be0r, W1)
    p1 = _spmm_sc(y1, src, dst, zeros_row)
    y2 = _b_call(p1, dinv, b1r, g1r, be1r, W2)
    p2 = _spmm_sc(y2, src, dst, zeros_row)
    return _c_call(p2, dinv, b2r)


# speedup vs baseline: 31.8403x; 1.0070x over previous
import functools

import jax
import jax.numpy as jnp
from jax import lax
from jax.experimental import pallas as pl
from jax.experimental.pallas import tpu as pltpu
from jax.experimental.pallas import tpu_sc as plsc

N = 10000
D = 128
E = 320000

NC = 2
NS = 16
NW = NC * NS
EPW = E // NW
CHUNK = 80
STEPS = EPW // CHUNK
NPAD = 10240
RPT = NPAD // NS
DEGW = 16

_BN_SCALE = 1.0 / (1.0 + 1e-5) ** 0.5

_sc_mesh = plsc.VectorSubcoreMesh(
    core_axis_name="c", subcore_axis_name="s", num_cores=NC, num_subcores=NS)


_DEG_GRP = 5


@functools.partial(
    pl.kernel,
    out_type=jax.ShapeDtypeStruct((NC, NPAD), jnp.float32),
    mesh=_sc_mesh,
    scratch_types=[
        pltpu.VMEM((STEPS, CHUNK), jnp.int32),
        pltpu.VMEM((CHUNK,), jnp.float32),
        pltpu.SemaphoreType.DMA,
        pltpu.SemaphoreType.DMA,
        pltpu.VMEM_SHARED((NPAD,), jnp.float32),
    ],
)
def _deg_sc(adj_hbm, zeros_hbm, out_hbm, didx, ones_v, isem, ssem, acc):
    c = lax.axis_index("c")
    s = lax.axis_index("s")
    w = s * NC + c

    pltpu.async_copy(adj_hbm.at[1, w], didx, isem)

    def fill(i, carry):
        ones_v[pl.ds(i * 16, 16)] = jnp.ones((16,), jnp.float32)
        return carry

    lax.fori_loop(0, CHUNK // 16, fill, 0)
    pltpu.sync_copy(zeros_hbm, acc.at[pl.ds(s * RPT, RPT)])
    pltpu.make_async_copy(adj_hbm.at[1, w], didx, isem).wait()
    plsc.subcore_barrier()

    def group(g, carry):
        for b in range(_DEG_GRP):
            pltpu.async_copy(ones_v, acc.at[didx.at[g * _DEG_GRP + b]], ssem,
                             add=True)
        for b in range(_DEG_GRP):
            pltpu.make_async_copy(ones_v, acc.at[didx.at[0]], ssem).wait()
        return carry

    lax.fori_loop(0, STEPS // _DEG_GRP, group, 0)
    plsc.subcore_barrier()
    pltpu.sync_copy(acc.at[pl.ds(s * RPT, RPT)], out_hbm.at[c, pl.ds(s * RPT, RPT)])


NBUF = 3


@functools.partial(
    pl.kernel,
    out_type=jax.ShapeDtypeStruct((NC, NPAD, D), jnp.float32),
    mesh=_sc_mesh,
    scratch_types=[
        pltpu.VMEM((STEPS, CHUNK), jnp.int32),
        pltpu.VMEM((NBUF, CHUNK), jnp.int32),
        pltpu.VMEM((NBUF, CHUNK, D), jnp.float32),
        pltpu.SemaphoreType.DMA,
        pltpu.SemaphoreType.DMA,
        pltpu.SemaphoreType.DMA,
        pltpu.SemaphoreType.DMA,
        pltpu.SemaphoreType.DMA,
        pltpu.SemaphoreType.DMA,
        pltpu.SemaphoreType.DMA,
        pltpu.SemaphoreType.DMA,
        pltpu.SemaphoreType.DMA,
        pltpu.VMEM_SHARED((NPAD, D), jnp.float32),
    ],
)
def _spmm_sc(y_hbm, adj_hbm, dst_hbm, zeros_hbm, out_hbm,
             sidx, dring, rows, g0, g1, g2, s0, s1, s2, d0, d1, d2, acc):
    c = lax.axis_index("c")
    s = lax.axis_index("s")
    w = s * NC + c
    gsems = (g0, g1, g2)
    ssems = (s0, s1, s2)
    dsems = (d0, d1, d2)
    pltpu.async_copy(adj_hbm.at[0, w], sidx, g0)
    init_src = y_hbm.at[pl.ds(s * RPT, RPT)]
    zero_src = zeros_hbm.at[pl.ds(0, RPT)]
    acc_dst = acc.at[pl.ds(s * RPT, RPT)]

    @pl.when(c == 0)
    def _():
        pltpu.async_copy(init_src, acc_dst, s0)

    @pl.when(c != 0)
    def _():
        pltpu.async_copy(zero_src, acc_dst, s0)

    pltpu.make_async_copy(adj_hbm.at[0, w], sidx, g0).wait()
    pltpu.make_async_copy(init_src, acc_dst, s0).wait()
    plsc.subcore_barrier()

    def sidx_at(j):
        return sidx.at[j]

    def didx_src(j):
        return dst_hbm.at[pl.ds(w * EPW + j * CHUNK, CHUNK)]

    def start_fetch(j, b):
        pltpu.async_copy(didx_src(j), dring.at[b], dsems[b])
        pltpu.async_copy(y_hbm.at[sidx_at(j)], rows.at[b], gsems[b])

    def wait_gather(j, b):
        pltpu.make_async_copy(y_hbm.at[sidx_at(j)], rows.at[b],
                              gsems[b]).wait()
        pltpu.make_async_copy(didx_src(j), dring.at[b], dsems[b]).wait()

    def start_scatter(b):
        pltpu.async_copy(rows.at[b], acc.at[dring.at[b]], ssems[b], add=True)

    def wait_scatter(b):
        pltpu.make_async_copy(rows.at[b], acc.at[dring.at[b]],
                              ssems[b]).wait()

    start_fetch(0, 0)
    start_fetch(1, 1)

    def outer(g, carry):
        for k in range(NBUF):
            j = g * NBUF + k
            bp = (k + NBUF - 1) % NBUF
            wait_gather(j, k)
            start_scatter(k)

            @pl.when(j >= 1)
            def _():
                wait_scatter(bp)

            start_fetch(j + 2, bp)
        return carry

    lax.fori_loop(0, STEPS // NBUF, outer, 0)
    for j in range((STEPS // NBUF) * NBUF, STEPS):
        k = j % NBUF
        bp = (k + NBUF - 1) % NBUF
        wait_gather(j, k)
        start_scatter(k)
        wait_scatter(bp)
    wait_scatter((STEPS - 1) % NBUF)
    plsc.subcore_barrier()
    pltpu.sync_copy(acc.at[pl.ds(s * RPT, RPT)], out_hbm.at[c, pl.ds(s * RPT, RPT)])


_R = 2048


def _a_body(degp_ref, x_ref, w_ref, y_ref, dinv_ref):
    deg = 1.0 + degp_ref[0] + degp_ref[1]
    dinv = lax.rsqrt(deg)
    dinv_ref[...] = dinv
    y_ref[...] = jnp.dot(x_ref[...], w_ref[...],
                         preferred_element_type=jnp.float32) * dinv


def _b_body(p_ref, dinv_ref, b_ref, g_ref, be_ref, w_ref, yn_ref):
    dinv = dinv_ref[...]
    z = (p_ref[0] + p_ref[1]) * dinv + b_ref[...]
    t = jnp.maximum(z * (g_ref[...] * _BN_SCALE) + be_ref[...], 0.0)
    yn_ref[...] = jnp.dot(t, w_ref[...],
                          preferred_element_type=jnp.float32) * dinv


def _c_body(p_ref, dinv_ref, b_ref, o_ref):
    z = (p_ref[0] + p_ref[1]) * dinv_ref[...] + b_ref[...]
    m = jnp.max(z, axis=1, keepdims=True)
    lse = jnp.log(jnp.sum(jnp.exp(z - m), axis=1, keepdims=True)) + m
    o_ref[...] = z - lse


_a_call = pl.pallas_call(
    _a_body,
    grid=(NPAD // _R,),
    in_specs=[
        pl.BlockSpec((NC, _R, 1), lambda i: (0, i, 0)),
        pl.BlockSpec((_R, D), lambda i: (i, 0)),
        pl.BlockSpec((D, D), lambda i: (0, 0)),
    ],
    out_specs=[
        pl.BlockSpec((_R, D), lambda i: (i, 0)),
        pl.BlockSpec((_R, 1), lambda i: (i, 0)),
    ],
    out_shape=[
        jax.ShapeDtypeStruct((NPAD, D), jnp.float32),
        jax.ShapeDtypeStruct((NPAD, 1), jnp.float32),
    ],
)

_b_call = pl.pallas_call(
    _b_body,
    grid=(NPAD // _R,),
    in_specs=[
        pl.BlockSpec((NC, _R, D), lambda i: (0, i, 0)),
        pl.BlockSpec((_R, 1), lambda i: (i, 0)),
        pl.BlockSpec((1, D), lambda i: (0, 0)),
        pl.BlockSpec((1, D), lambda i: (0, 0)),
        pl.BlockSpec((1, D), lambda i: (0, 0)),
        pl.BlockSpec((D, D), lambda i: (0, 0)),
    ],
    out_specs=pl.BlockSpec((_R, D), lambda i: (i, 0)),
    out_shape=jax.ShapeDtypeStruct((NPAD, D), jnp.float32),
)

_c_call = pl.pallas_call(
    _c_body,
    grid=(NPAD // _R,),
    in_specs=[
        pl.BlockSpec((NC, _R, D), lambda i: (0, i, 0)),
        pl.BlockSpec((_R, 1), lambda i: (i, 0)),
        pl.BlockSpec((1, D), lambda i: (0, 0)),
    ],
    out_specs=pl.BlockSpec((_R, D), lambda i: (i, 0)),
    out_shape=jax.ShapeDtypeStruct((N, D), jnp.float32),
)


def kernel(x, adj_t, W0, b0, g0, be0, W1, b1, g1, be1, W2, b2):
    adj_i = adj_t.astype(jnp.int32)
    adj_r = adj_i.reshape(2, NW, STEPS, CHUNK)
    dst = adj_i[1]
    zeros_deg = jnp.zeros((RPT,), jnp.float32)
    zeros_row = jnp.zeros((RPT, D), jnp.float32)
    b0r, g0r, be0r = b0.reshape(1, D), g0.reshape(1, D), be0.reshape(1, D)
    b1r, g1r, be1r = b1.reshape(1, D), g1.reshape(1, D), be1.reshape(1, D)
    b2r = b2.reshape(1, D)

    degp = _deg_sc(adj_r, zeros_deg).reshape(NC, NPAD, 1)
    y0, dinv = _a_call(degp, x, W0)
    p0 = _spmm_sc(y0, adj_r, dst, zeros_row)
    y1 = _b_call(p0, dinv, b0r, g0r, be0r, W1)
    p1 = _spmm_sc(y1, adj_r, dst, zeros_row)
    y2 = _b_call(p1, dinv, b1r, g1r, be1r, W2)
    p2 = _spmm_sc(y2, adj_r, dst, zeros_row)
    return _c_call(p2, dinv, b2r)


# speedup vs baseline: 32.4179x; 1.0181x over previous
import functools

import jax
import jax.numpy as jnp
from jax import lax
from jax.experimental import pallas as pl
from jax.experimental.pallas import tpu as pltpu
from jax.experimental.pallas import tpu_sc as plsc

N = 10000
D = 128
E = 320000

NC = 2
NS = 16
NW = NC * NS
EPW = E // NW
CHUNK = 80
STEPS = EPW // CHUNK
NPAD = 10240
RPT = NPAD // NS
DEGW = 16

_BN_SCALE = 1.0 / (1.0 + 1e-5) ** 0.5

_sc_mesh = plsc.VectorSubcoreMesh(
    core_axis_name="c", subcore_axis_name="s", num_cores=NC, num_subcores=NS)


_DEG_GRP = 5


@functools.partial(
    pl.kernel,
    out_type=jax.ShapeDtypeStruct((NC, NPAD), jnp.float32),
    mesh=_sc_mesh,
    scratch_types=[
        pltpu.VMEM((STEPS, CHUNK), jnp.int32),
        pltpu.VMEM((CHUNK,), jnp.float32),
        pltpu.SemaphoreType.DMA,
        pltpu.SemaphoreType.DMA,
        pltpu.VMEM_SHARED((NPAD,), jnp.float32),
    ],
)
def _deg_sc(adj_hbm, zeros_hbm, out_hbm, didx, ones_v, isem, ssem, acc):
    c = lax.axis_index("c")
    s = lax.axis_index("s")
    w = s * NC + c

    pltpu.async_copy(adj_hbm.at[1, w], didx, isem)

    def fill(i, carry):
        ones_v[pl.ds(i * 16, 16)] = jnp.ones((16,), jnp.float32)
        return carry

    lax.fori_loop(0, CHUNK // 16, fill, 0)
    pltpu.sync_copy(zeros_hbm, acc.at[pl.ds(s * RPT, RPT)])
    pltpu.make_async_copy(adj_hbm.at[1, w], didx, isem).wait()
    plsc.subcore_barrier()

    def group(g, carry):
        for b in range(_DEG_GRP):
            pltpu.async_copy(ones_v, acc.at[didx.at[g * _DEG_GRP + b]], ssem,
                             add=True)
        for b in range(_DEG_GRP):
            pltpu.make_async_copy(ones_v, acc.at[didx.at[0]], ssem).wait()
        return carry

    lax.fori_loop(0, STEPS // _DEG_GRP, group, 0)
    plsc.subcore_barrier()
    pltpu.sync_copy(acc.at[pl.ds(s * RPT, RPT)], out_hbm.at[c, pl.ds(s * RPT, RPT)])


NBUF = 3


@functools.partial(
    pl.kernel,
    out_type=jax.ShapeDtypeStruct((NC, NPAD, D), jnp.float32),
    mesh=_sc_mesh,
    scratch_types=[
        pltpu.VMEM((STEPS, CHUNK), jnp.int32),
        pltpu.VMEM((NBUF, CHUNK), jnp.int32),
        pltpu.VMEM((NBUF, CHUNK, D), jnp.float32),
        pltpu.SemaphoreType.DMA,
        pltpu.SemaphoreType.DMA,
        pltpu.SemaphoreType.DMA,
        pltpu.SemaphoreType.DMA,
        pltpu.SemaphoreType.DMA,
        pltpu.SemaphoreType.DMA,
        pltpu.SemaphoreType.DMA,
        pltpu.SemaphoreType.DMA,
        pltpu.SemaphoreType.DMA,
        pltpu.VMEM_SHARED((NPAD, D), jnp.float32),
    ],
)
def _spmm_sc(y_hbm, adj_hbm, dst_hbm, zeros_hbm, out_hbm,
             sidx, dring, rows, g0, g1, g2, s0, s1, s2, d0, d1, d2, acc):
    c = lax.axis_index("c")
    s = lax.axis_index("s")
    w = s * NC + c
    gsems = (g0, g1, g2)
    ssems = (s0, s1, s2)
    dsems = (d0, d1, d2)
    pltpu.async_copy(adj_hbm.at[0, w], sidx, g0)
    init_src = y_hbm.at[pl.ds(s * RPT, RPT)]
    zero_src = zeros_hbm.at[pl.ds(0, RPT)]
    acc_dst = acc.at[pl.ds(s * RPT, RPT)]

    @pl.when(c == 0)
    def _():
        pltpu.async_copy(init_src, acc_dst, s0)

    @pl.when(c != 0)
    def _():
        pltpu.async_copy(zero_src, acc_dst, s0)

    pltpu.make_async_copy(adj_hbm.at[0, w], sidx, g0).wait()
    pltpu.make_async_copy(init_src, acc_dst, s0).wait()
    plsc.subcore_barrier()

    def sidx_at(j):
        return sidx.at[j]

    def didx_src(j):
        return dst_hbm.at[pl.ds(w * EPW + j * CHUNK, CHUNK)]

    def start_fetch(j, b):
        pltpu.async_copy(didx_src(j), dring.at[b], dsems[b])
        pltpu.async_copy(y_hbm.at[sidx_at(j)], rows.at[b], gsems[b])

    def wait_gather(j, b):
        pltpu.make_async_copy(y_hbm.at[sidx_at(j)], rows.at[b],
                              gsems[b]).wait()
        pltpu.make_async_copy(didx_src(j), dring.at[b], dsems[b]).wait()

    def start_scatter(b):
        pltpu.async_copy(rows.at[b], acc.at[dring.at[b]], ssems[b], add=True)

    def wait_scatter(b):
        pltpu.make_async_copy(rows.at[b], acc.at[dring.at[b]],
                              ssems[b]).wait()

    start_fetch(0, 0)
    start_fetch(1, 1)

    def outer(g, carry):
        for k in range(NBUF):
            j = g * NBUF + k
            bp = (k + NBUF - 1) % NBUF
            wait_gather(j, k)
            start_scatter(k)

            @pl.when(j >= 1)
            def _():
                wait_scatter(bp)

            start_fetch(j + 2, bp)
        return carry

    lax.fori_loop(0, STEPS // NBUF, outer, 0)
    for j in range((STEPS // NBUF) * NBUF, STEPS):
        k = j % NBUF
        bp = (k + NBUF - 1) % NBUF
        wait_gather(j, k)
        start_scatter(k)
        wait_scatter(bp)
    wait_scatter((STEPS - 1) % NBUF)
    plsc.subcore_barrier()
    pltpu.sync_copy(acc.at[pl.ds(s * RPT, RPT)], out_hbm.at[c, pl.ds(s * RPT, RPT)])


_R = 5120


def _a_body(degp_ref, x_ref, w_ref, y_ref, dinv_ref):
    deg = 1.0 + degp_ref[0] + degp_ref[1]
    dinv = lax.rsqrt(deg)
    dinv_ref[...] = dinv
    y_ref[...] = jnp.dot(x_ref[...], w_ref[...],
                         preferred_element_type=jnp.float32) * dinv


def _b_body(p_ref, dinv_ref, b_ref, g_ref, be_ref, w_ref, yn_ref):
    dinv = dinv_ref[...]
    z = (p_ref[0] + p_ref[1]) * dinv + b_ref[...]
    t = jnp.maximum(z * (g_ref[...] * _BN_SCALE) + be_ref[...], 0.0)
    yn_ref[...] = jnp.dot(t, w_ref[...],
                          preferred_element_type=jnp.float32) * dinv


def _c_body(p_ref, dinv_ref, b_ref, o_ref):
    z = (p_ref[0] + p_ref[1]) * dinv_ref[...] + b_ref[...]
    m = jnp.max(z, axis=1, keepdims=True)
    lse = jnp.log(jnp.sum(jnp.exp(z - m), axis=1, keepdims=True)) + m
    o_ref[...] = z - lse


_a_call = pl.pallas_call(
    _a_body,
    grid=(NPAD // _R,),
    in_specs=[
        pl.BlockSpec((NC, _R, 1), lambda i: (0, i, 0)),
        pl.BlockSpec((_R, D), lambda i: (i, 0)),
        pl.BlockSpec((D, D), lambda i: (0, 0)),
    ],
    out_specs=[
        pl.BlockSpec((_R, D), lambda i: (i, 0)),
        pl.BlockSpec((_R, 1), lambda i: (i, 0)),
    ],
    out_shape=[
        jax.ShapeDtypeStruct((NPAD, D), jnp.float32),
        jax.ShapeDtypeStruct((NPAD, 1), jnp.float32),
    ],
)

_b_call = pl.pallas_call(
    _b_body,
    grid=(NPAD // _R,),
    in_specs=[
        pl.BlockSpec((NC, _R, D), lambda i: (0, i, 0)),
        pl.BlockSpec((_R, 1), lambda i: (i, 0)),
        pl.BlockSpec((1, D), lambda i: (0, 0)),
        pl.BlockSpec((1, D), lambda i: (0, 0)),
        pl.BlockSpec((1, D), lambda i: (0, 0)),
        pl.BlockSpec((D, D), lambda i: (0, 0)),
    ],
    out_specs=pl.BlockSpec((_R, D), lambda i: (i, 0)),
    out_shape=jax.ShapeDtypeStruct((NPAD, D), jnp.float32),
)

_c_call = pl.pallas_call(
    _c_body,
    grid=(NPAD // _R,),
    in_specs=[
        pl.BlockSpec((NC, _R, D), lambda i: (0, i, 0)),
        pl.BlockSpec((_R, 1), lambda i: (i, 0)),
        pl.BlockSpec((1, D), lambda i: (0, 0)),
    ],
    out_specs=pl.BlockSpec((_R, D), lambda i: (i, 0)),
    out_shape=jax.ShapeDtypeStruct((N, D), jnp.float32),
)


def kernel(x, adj_t, W0, b0, g0, be0, W1, b1, g1, be1, W2, b2):
    adj_i = adj_t.astype(jnp.int32)
    adj_r = adj_i.reshape(2, NW, STEPS, CHUNK)
    dst = adj_i[1]
    zeros_deg = jnp.zeros((RPT,), jnp.float32)
    zeros_row = jnp.zeros((RPT, D), jnp.float32)
    b0r, g0r, be0r = b0.reshape(1, D), g0.reshape(1, D), be0.reshape(1, D)
    b1r, g1r, be1r = b1.reshape(1, D), g1.reshape(1, D), be1.reshape(1, D)
    b2r = b2.reshape(1, D)

    degp = _deg_sc(adj_r, zeros_deg).reshape(NC, NPAD, 1)
    y0, dinv = _a_call(degp, x, W0)
    p0 = _spmm_sc(y0, adj_r, dst, zeros_row)
    y1 = _b_call(p0, dinv, b0r, g0r, be0r, W1)
    p1 = _spmm_sc(y1, adj_r, dst, zeros_row)
    y2 = _b_call(p1, dinv, b1r, g1r, be1r, W2)
    p2 = _spmm_sc(y2, adj_r, dst, zeros_row)
    return _c_call(p2, dinv, b2r)
